# baseline (speedup 1.0000x reference)
import functools

import jax
import jax.numpy as jnp
from jax import lax
from jax.experimental import pallas as pl
from jax.experimental.pallas import tpu as pltpu
from jax.experimental.pallas import tpu_sc as plsc

N = 10000
E = 320000
D = 128

NC = 2
NS = 16
NW = NC * NS
CHUNK = 128
NCH = E // CHUNK
GU = 12
LOOP = 84
NROW = 3
NIDX = 4

DEG_PAD = 10240
RBLK = 128
NRBLK = N // RBLK

_SC_MESH = plsc.VectorSubcoreMesh(
    core_axis_name="c", subcore_axis_name="s", num_cores=NC, num_subcores=NS)

_f32 = jnp.float32


CPT = 80


@functools.partial(
    pl.kernel,
    out_type=jax.ShapeDtypeStruct((NC * DEG_PAD,), _f32),
    mesh=_SC_MESH,
    scratch_types=[
        pltpu.VMEM((CPT, CHUNK), jnp.int32),
        pltpu.VMEM((CHUNK,), _f32),
        pltpu.VMEM((DEG_PAD // NS,), _f32),
        pltpu.VMEM_SHARED((DEG_PAD,), _f32),
        pltpu.SemaphoreType.DMA,
        pltpu.SemaphoreType.DMA,
    ],
)
def _sc_deg(dst_hbm, deg_out, dstall, onesv, zv, deg_sh, isem, ssem):
    c = lax.axis_index("c")
    s = lax.axis_index("s")
    wid = s * NC + c
    nreal = jnp.clip(NCH - wid * CPT, 0, CPT)

    pltpu.async_copy(dst_hbm.at[wid], dstall, isem)

    one = jnp.full((16,), 1.0, _f32)
    for i in range(CHUNK // 16):
        onesv[pl.ds(i * 16, 16)] = one
    zero = jnp.zeros((16,), _f32)

    def zfill(i, _):
        zv[pl.ds(i * 16, 16)] = zero
        return 0

    lax.fori_loop(0, DEG_PAD // NS // 16, zfill, 0, unroll=4)

    pltpu.sync_copy(zv, deg_sh.at[pl.ds(s * (DEG_PAD // NS), DEG_PAD // NS)])
    pltpu.make_async_copy(dst_hbm.at[0], dstall, isem).wait()
    plsc.subcore_barrier()

    def fire(i, _):
        pltpu.async_copy(onesv, deg_sh.at[dstall.at[i]], ssem, add=True)
        return 0

    lax.fori_loop(0, nreal, fire, 0)

    def drain(i, _):
        pltpu.make_async_copy(dst_hbm.at[0, 0], dstall.at[0], ssem).wait()
        return 0

    lax.fori_loop(0, nreal, drain, 0)
    plsc.subcore_barrier()
    pltpu.sync_copy(
        deg_sh.at[pl.ds(s * (DEG_PAD // NS), DEG_PAD // NS)],
        deg_out.at[pl.ds(c * DEG_PAD + s * (DEG_PAD // NS), DEG_PAD // NS)])


@functools.partial(
    pl.kernel,
    out_type=jax.ShapeDtypeStruct((NC * N, D), _f32),
    mesh=_SC_MESH,
    scratch_types=[
        [pltpu.VMEM((CHUNK, D), _f32)] * NROW,
        [pltpu.VMEM((2, CHUNK), jnp.int32)] * NIDX,
        [pltpu.SemaphoreType.DMA] * NROW,
        [pltpu.SemaphoreType.DMA] * NIDX,
        [pltpu.SemaphoreType.DMA] * NROW,
        pltpu.VMEM_SHARED((N, D), _f32),
    ],
)
def _sc_agg(edg_hbm, y_hbm, acc_out, rows, idxb, gsem, isem, ssem, acc_sh):
    c = lax.axis_index("c")
    s = lax.axis_index("s")
    wid = s * NC + c
    nreal = (NCH - 1 - wid) // NW + 1

    def fire_idx(cc, k):
        pltpu.async_copy(edg_hbm.at[wid + cc * NW], idxb[k], isem[k])

    def drain_idx(k):
        pltpu.make_async_copy(edg_hbm.at[0], idxb[k], isem[k]).wait()

    def fire_gather(k, b):
        pltpu.async_copy(y_hbm.at[idxb[k].at[0]], rows[b], gsem[b])

    def drain_gather(b):
        pltpu.make_async_copy(y_hbm.at[pl.ds(0, CHUNK)], rows[b],
                              gsem[b]).wait()

    def fire_scat(k, b):
        pltpu.async_copy(rows[b], acc_sh.at[idxb[k].at[1]], ssem[b],
                         add=True)

    def drain_scat(b):
        pltpu.make_async_copy(y_hbm.at[pl.ds(0, CHUNK)], rows[b],
                              ssem[b]).wait()

    fire_idx(0, 0)
    fire_idx(1, 1)

    zero = jnp.zeros((16,), _f32)
    zb = rows[NROW - 1]

    def zbody(i, _):
        zb[i >> 3, pl.ds((i & 7) * 16, 16)] = zero
        return 0

    lax.fori_loop(0, CHUNK * (D // 16), zbody, 0, unroll=8)

    nrb = 4 + jnp.where(s < NRBLK - 4 * NS, 1, 0)

    def zcopy(i, _):
        pltpu.sync_copy(zb, acc_sh.at[pl.ds((s + i * NS) * RBLK, RBLK)])
        return 0

    lax.fori_loop(0, nrb, zcopy, 0)

    @pl.when(s == NS - 1)
    def _():
        pltpu.sync_copy(zb.at[pl.ds(0, N - NRBLK * RBLK)],
                        acc_sh.at[pl.ds(NRBLK * RBLK, N - NRBLK * RBLK)])

    plsc.subcore_barrier()

    drain_idx(0)
    fire_gather(0, 0)

    def body(g, _):
        for j in range(GU):
            cc = g * GU + j
            alive0 = cc < nreal
            alive1 = cc + 1 < nreal
            alive2 = cc + 2 < nreal

            @pl.when((cc >= 2) & alive0)
            def _(j=j):
                drain_scat((j - 2) % NROW)

            @pl.when(alive2)
            def _(cc=cc, j=j):
                fire_idx(cc + 2, (j + 2) % NIDX)

            @pl.when(alive1)
            def _(j=j):
                drain_idx((j + 1) % NIDX)
                fire_gather((j + 1) % NIDX, (j + 1) % NROW)

            @pl.when(alive0)
            def _(j=j):
                drain_gather(j % NROW)
                fire_scat(j % NIDX, j % NROW)
        return 0

    lax.fori_loop(0, LOOP // GU, body, 0)

    tmax = nreal - 1
    for b in range(NROW):
        @pl.when((tmax % NROW == b) | ((tmax - 1) % NROW == b))
        def _(b=b):
            drain_scat(b)

    plsc.subcore_barrier()

    def wcopy(i, _):
        r0 = (s + i * NS) * RBLK
        pltpu.sync_copy(acc_sh.at[pl.ds(r0, RBLK)],
                        acc_out.at[pl.ds(c * N + r0, RBLK)])
        return 0

    lax.fori_loop(0, nrb, wcopy, 0)

    @pl.when(s == NS - 1)
    def _():
        tail = N - NRBLK * RBLK
        pltpu.sync_copy(acc_sh.at[pl.ds(NRBLK * RBLK, tail)],
                        acc_out.at[pl.ds(c * N + NRBLK * RBLK, tail)])


_BLK = 2000
_GRID = N // _BLK


def _prep_body(nodes_ref, w_ref, ds_ref, y_ref):
    dinv = lax.rsqrt(ds_ref[...] + 1.0)
    x = lax.dot_general(nodes_ref[...], w_ref[...],
                        (((1,), (1,)), ((), ())),
                        preferred_element_type=_f32)
    y_ref[...] = x * dinv


_tc_prep = pl.pallas_call(
    _prep_body,
    grid=(_GRID,),
    in_specs=[
        pl.BlockSpec((_BLK, D), lambda i: (i, 0)),
        pl.BlockSpec((D, D), lambda i: (0, 0)),
        pl.BlockSpec((_BLK, 1), lambda i: (i, 0)),
    ],
    out_specs=pl.BlockSpec((_BLK, D), lambda i: (i, 0)),
    out_shape=jax.ShapeDtypeStruct((N, D), _f32),
)


def _post_body(a0_ref, a1_ref, y_ref, ds_ref, b_ref, g_ref, be_ref, o_ref):
    dinv = lax.rsqrt(ds_ref[...] + 1.0)
    pre = (a0_ref[...] + a1_ref[...] + y_ref[...]) * dinv + b_ref[...]
    mu = jnp.mean(pre, axis=-1, keepdims=True)
    dev = pre - mu
    var = jnp.mean(dev * dev, axis=-1, keepdims=True)
    o = dev * lax.rsqrt(var + 1e-5) * g_ref[...] + be_ref[...]
    o_ref[...] = jnp.maximum(o, 0.0)


_tc_post = pl.pallas_call(
    _post_body,
    grid=(_GRID,),
    in_specs=[
        pl.BlockSpec((_BLK, D), lambda i: (i, 0)),
        pl.BlockSpec((_BLK, D), lambda i: (i + _GRID, 0)),
        pl.BlockSpec((_BLK, D), lambda i: (i, 0)),
        pl.BlockSpec((_BLK, 1), lambda i: (i, 0)),
        pl.BlockSpec((1, D), lambda i: (0, 0)),
        pl.BlockSpec((1, D), lambda i: (0, 0)),
        pl.BlockSpec((1, D), lambda i: (0, 0)),
    ],
    out_specs=pl.BlockSpec((_BLK, D), lambda i: (i, 0)),
    out_shape=jax.ShapeDtypeStruct((N, D), _f32),
)


def kernel(nodes, edges, W, b, gamma, beta):
    e = edges.astype(jnp.int32)
    edg = jnp.stack([e[0].reshape(NCH, CHUNK), e[1].reshape(NCH, CHUNK)],
                    axis=1)
    dst3 = jnp.concatenate(
        [e[1].reshape(NCH, CHUNK),
         jnp.zeros((CPT * NW - NCH, CHUNK), jnp.int32)]).reshape(
             NW, CPT, CHUNK)

    degp = _sc_deg(dst3)
    dsum = (degp[:N] + degp[DEG_PAD:DEG_PAD + N]).reshape(N, 1)

    y = _tc_prep(nodes, W, dsum)
    accp = _sc_agg(edg, y)
    out = _tc_post(accp, accp, y, dsum,
                   b.reshape(1, D), gamma.reshape(1, D), beta.reshape(1, D))
    return out


# speedup vs baseline: 1.0563x; 1.0563x over previous
import functools

import jax
import jax.numpy as jnp
from jax import lax
from jax.experimental import pallas as pl
from jax.experimental.pallas import tpu as pltpu
from jax.experimental.pallas import tpu_sc as plsc

N = 10000
E = 320000
D = 128

NC = 2
NS = 16
NW = NC * NS
CHUNK = 128
NCH = E // CHUNK
GU = 12
LOOP = 84
NROW = 3
NIDX = 4

DEG_PAD = 10240
RBLK = 128
NRBLK = N // RBLK

_SC_MESH = plsc.VectorSubcoreMesh(
    core_axis_name="c", subcore_axis_name="s", num_cores=NC, num_subcores=NS)

_f32 = jnp.float32


CPT = 80


@functools.partial(
    pl.kernel,
    out_type=jax.ShapeDtypeStruct((NC * DEG_PAD,), _f32),
    mesh=_SC_MESH,
    scratch_types=[
        pltpu.VMEM((CPT, 2, CHUNK), jnp.int32),
        pltpu.VMEM((CHUNK,), _f32),
        pltpu.VMEM((DEG_PAD // NS,), _f32),
        pltpu.VMEM_SHARED((DEG_PAD,), _f32),
        pltpu.SemaphoreType.DMA,
        pltpu.SemaphoreType.DMA,
    ],
)
def _sc_deg(edg_hbm, deg_out, edgall, onesv, zv, deg_sh, isem, ssem):
    c = lax.axis_index("c")
    s = lax.axis_index("s")
    wid = s * NC + c
    nreal = jnp.clip(NCH - wid * CPT, 0, CPT)

    pltpu.async_copy(edg_hbm.at[wid], edgall, isem)

    one = jnp.full((16,), 1.0, _f32)
    for i in range(CHUNK // 16):
        onesv[pl.ds(i * 16, 16)] = one
    zero = jnp.zeros((16,), _f32)

    def zfill(i, _):
        zv[pl.ds(i * 16, 16)] = zero
        return 0

    lax.fori_loop(0, DEG_PAD // NS // 16, zfill, 0, unroll=4)

    pltpu.sync_copy(zv, deg_sh.at[pl.ds(s * (DEG_PAD // NS), DEG_PAD // NS)])
    pltpu.make_async_copy(edg_hbm.at[0], edgall, isem).wait()
    plsc.subcore_barrier()

    def fire(i, _):
        pltpu.async_copy(onesv, deg_sh.at[edgall.at[i, 1]], ssem, add=True)
        return 0

    lax.fori_loop(0, nreal, fire, 0)

    def drain(i, _):
        pltpu.make_async_copy(edg_hbm.at[0, 0, 1], edgall.at[0, 1],
                              ssem).wait()
        return 0

    lax.fori_loop(0, nreal, drain, 0)
    plsc.subcore_barrier()
    pltpu.sync_copy(
        deg_sh.at[pl.ds(s * (DEG_PAD // NS), DEG_PAD // NS)],
        deg_out.at[pl.ds(c * DEG_PAD + s * (DEG_PAD // NS), DEG_PAD // NS)])


@functools.partial(
    pl.kernel,
    out_type=jax.ShapeDtypeStruct((NC * N, D), _f32),
    mesh=_SC_MESH,
    scratch_types=[
        [pltpu.VMEM((CHUNK, D), _f32)] * NROW,
        [pltpu.VMEM((2, CHUNK), jnp.int32)] * NIDX,
        [pltpu.SemaphoreType.DMA] * NROW,
        [pltpu.SemaphoreType.DMA] * NIDX,
        [pltpu.SemaphoreType.DMA] * NROW,
        pltpu.VMEM_SHARED((N, D), _f32),
    ],
)
def _sc_agg(edg_hbm, y_hbm, acc_out, rows, idxb, gsem, isem, ssem, acc_sh):
    c = lax.axis_index("c")
    s = lax.axis_index("s")
    wid = s * NC + c
    nreal = (NCH - 1 - wid) // NW + 1

    def fire_idx(cc, k):
        pltpu.async_copy(edg_hbm.at[wid + cc * NW], idxb[k], isem[k])

    def drain_idx(k):
        pltpu.make_async_copy(edg_hbm.at[0], idxb[k], isem[k]).wait()

    def fire_gather(k, b):
        pltpu.async_copy(y_hbm.at[idxb[k].at[0]], rows[b], gsem[b])

    def drain_gather(b):
        pltpu.make_async_copy(y_hbm.at[pl.ds(0, CHUNK)], rows[b],
                              gsem[b]).wait()

    def fire_scat(k, b):
        pltpu.async_copy(rows[b], acc_sh.at[idxb[k].at[1]], ssem[b],
                         add=True)

    def drain_scat(b):
        pltpu.make_async_copy(y_hbm.at[pl.ds(0, CHUNK)], rows[b],
                              ssem[b]).wait()

    fire_idx(0, 0)
    fire_idx(1, 1)

    zero = jnp.zeros((16,), _f32)
    zb = rows[NROW - 1]

    def zbody(i, _):
        zb[i >> 3, pl.ds((i & 7) * 16, 16)] = zero
        return 0

    lax.fori_loop(0, CHUNK * (D // 16), zbody, 0, unroll=8)

    nrb = 4 + jnp.where(s < NRBLK - 4 * NS, 1, 0)

    def zcopy(i, _):
        pltpu.sync_copy(zb, acc_sh.at[pl.ds((s + i * NS) * RBLK, RBLK)])
        return 0

    lax.fori_loop(0, nrb, zcopy, 0)

    @pl.when(s == NS - 1)
    def _():
        pltpu.sync_copy(zb.at[pl.ds(0, N - NRBLK * RBLK)],
                        acc_sh.at[pl.ds(NRBLK * RBLK, N - NRBLK * RBLK)])

    plsc.subcore_barrier()

    drain_idx(0)
    fire_gather(0, 0)

    def body(g, _):
        for j in range(GU):
            cc = g * GU + j
            alive0 = cc < nreal
            alive1 = cc + 1 < nreal
            alive2 = cc + 2 < nreal

            @pl.when((cc >= 2) & alive0)
            def _(j=j):
                drain_scat((j - 2) % NROW)

            @pl.when(alive2)
            def _(cc=cc, j=j):
                fire_idx(cc + 2, (j + 2) % NIDX)

            @pl.when(alive1)
            def _(j=j):
                drain_idx((j + 1) % NIDX)
                fire_gather((j + 1) % NIDX, (j + 1) % NROW)

            @pl.when(alive0)
            def _(j=j):
                drain_gather(j % NROW)
                fire_scat(j % NIDX, j % NROW)
        return 0

    lax.fori_loop(0, LOOP // GU, body, 0)

    tmax = nreal - 1
    for b in range(NROW):
        @pl.when((tmax % NROW == b) | ((tmax - 1) % NROW == b))
        def _(b=b):
            drain_scat(b)

    plsc.subcore_barrier()

    def wcopy(i, _):
        r0 = (s + i * NS) * RBLK
        pltpu.sync_copy(acc_sh.at[pl.ds(r0, RBLK)],
                        acc_out.at[pl.ds(c * N + r0, RBLK)])
        return 0

    lax.fori_loop(0, nrb, wcopy, 0)

    @pl.when(s == NS - 1)
    def _():
        tail = N - NRBLK * RBLK
        pltpu.sync_copy(acc_sh.at[pl.ds(NRBLK * RBLK, tail)],
                        acc_out.at[pl.ds(c * N + NRBLK * RBLK, tail)])


_BLK = 2000
_GRID = N // _BLK


def _prep_body(nodes_ref, w_ref, ds_ref, y_ref):
    dinv = lax.rsqrt(ds_ref[...] + 1.0)
    x = lax.dot_general(nodes_ref[...], w_ref[...],
                        (((1,), (1,)), ((), ())),
                        preferred_element_type=_f32)
    y_ref[...] = x * dinv


_tc_prep = pl.pallas_call(
    _prep_body,
    grid=(_GRID,),
    in_specs=[
        pl.BlockSpec((_BLK, D), lambda i: (i, 0)),
        pl.BlockSpec((D, D), lambda i: (0, 0)),
        pl.BlockSpec((_BLK, 1), lambda i: (i, 0)),
    ],
    out_specs=pl.BlockSpec((_BLK, D), lambda i: (i, 0)),
    out_shape=jax.ShapeDtypeStruct((N, D), _f32),
)


def _post_body(a0_ref, a1_ref, y_ref, ds_ref, b_ref, g_ref, be_ref, o_ref):
    dinv = lax.rsqrt(ds_ref[...] + 1.0)
    pre = (a0_ref[...] + a1_ref[...] + y_ref[...]) * dinv + b_ref[...]
    mu = jnp.mean(pre, axis=-1, keepdims=True)
    dev = pre - mu
    var = jnp.mean(dev * dev, axis=-1, keepdims=True)
    o = dev * lax.rsqrt(var + 1e-5) * g_ref[...] + be_ref[...]
    o_ref[...] = jnp.maximum(o, 0.0)


_tc_post = pl.pallas_call(
    _post_body,
    grid=(_GRID,),
    in_specs=[
        pl.BlockSpec((_BLK, D), lambda i: (i, 0)),
        pl.BlockSpec((_BLK, D), lambda i: (i + _GRID, 0)),
        pl.BlockSpec((_BLK, D), lambda i: (i, 0)),
        pl.BlockSpec((_BLK, 1), lambda i: (i, 0)),
        pl.BlockSpec((1, D), lambda i: (0, 0)),
        pl.BlockSpec((1, D), lambda i: (0, 0)),
        pl.BlockSpec((1, D), lambda i: (0, 0)),
    ],
    out_specs=pl.BlockSpec((_BLK, D), lambda i: (i, 0)),
    out_shape=jax.ShapeDtypeStruct((N, D), _f32),
)


def kernel(nodes, edges, W, b, gamma, beta):
    e = edges.astype(jnp.int32)
    edgp = jnp.concatenate(
        [jnp.stack([e[0].reshape(NCH, CHUNK), e[1].reshape(NCH, CHUNK)],
                   axis=1),
         jnp.zeros((CPT * NW - NCH, 2, CHUNK), jnp.int32)])

    degp = _sc_deg(edgp.reshape(NW, CPT, 2, CHUNK))
    dsum = (degp[:N] + degp[DEG_PAD:DEG_PAD + N]).reshape(N, 1)

    y = _tc_prep(nodes, W, dsum)
    accp = _sc_agg(edgp, y)
    out = _tc_post(accp, accp, y, dsum,
                   b.reshape(1, D), gamma.reshape(1, D), beta.reshape(1, D))
    return out
